# baseline (speedup 1.0000x reference)
import jax
import jax.numpy as jnp
from jax.experimental import pallas as pl
from jax.experimental.pallas import tpu as pltpu

B = 16384
RB = 2048


def _body(pb_ref, ps_ref, pbm_ref, psm_ref,
          d_pb_ref, d_def_ref, d_nw_ref, d_w_ref,
          d_h1tt_ref, d_h1tt_off_ref, d_h1c_ref, d_h1c_on_ref, d_h1c_off_ref,
          d_h2tt_ref, d_h2tt_off_ref, d_h2c_ref, d_h2c_on_ref, d_h2c_off_ref,
          d_ps_ref, pb_out_ref, ps_out_ref):
    pb = pb_ref[...]
    ps = ps_ref[...]
    pbm = pbm_ref[...]
    psm = psm_ref[...]

    pb1 = pb * pbm + (1.0 - pbm) * d_pb_ref[...]

    cond_nw = (pbm[:, 0:1] > 0.5) & (pb1[:, 0:1] > 0.5)
    cond_w = (pbm[:, 1:2] > 0.5) & (pb1[:, 1:2] > 0.5)
    ht1_known = pbm[:, 2:3] > 0.5
    ht1_hot = pb1[:, 2:3] > 0.5
    ht1_on = ht1_known & ht1_hot
    ht1_off = ht1_known & (~ht1_hot)
    ht2_known = pbm[:, 6:7] > 0.5
    ht2_hot = pb1[:, 6:7] > 0.5
    ht2_on = ht2_known & ht2_hot
    ht2_off = ht2_known & (~ht2_hot)

    def_fill = jnp.where(cond_w, d_w_ref[...],
                         jnp.where(cond_nw, d_nw_ref[...], d_def_ref[...]))
    ht1_tt = jnp.where(ht1_off, d_h1tt_off_ref[...], d_h1tt_ref[...])
    ht2_tt = jnp.where(ht2_off, d_h2tt_off_ref[...], d_h2tt_ref[...])
    ht1_cool = jnp.where(ht1_off, d_h1c_off_ref[...],
                         jnp.where(ht1_on, d_h1c_on_ref[...], d_h1c_ref[...]))
    ht2_cool = jnp.where(ht2_off, d_h2c_off_ref[...],
                         jnp.where(ht2_on, d_h2c_on_ref[...], d_h2c_ref[...]))

    rb = pb.shape[0]
    fill_pb = jnp.concatenate(
        [jnp.zeros((rb, 3), jnp.float32), ht1_cool,
         jnp.zeros((rb, 1), jnp.float32), ht2_cool], axis=1)
    col = jax.lax.broadcasted_iota(jnp.int32, (1, 9), 1)
    filled = ((col >= 3) & (col < 6)) | (col >= 7)
    pb2 = pb1 * pbm + (1.0 - pbm) * fill_pb
    pb_out_ref[...] = jnp.where(filled, pb2, pb1)

    fill_ps = jnp.concatenate([def_fill, ht1_tt, ht2_tt], axis=1)
    t = ps * psm + (1.0 - psm) * fill_ps
    ps_out_ref[...] = t * psm + (1.0 - psm) * d_ps_ref[...]


def kernel(proc_bool, proc_scalar, proc_bool_mask, proc_scalar_mask,
           p_pb_def, p_def_def, p_def_nw, p_def_w,
           p_ht1_tt_def, p_ht1_tt_off,
           p_ht1_cool_def, p_ht1_cool_on, p_ht1_cool_off,
           p_ht2_tt_def, p_ht2_tt_off,
           p_ht2_cool_def, p_ht2_cool_on, p_ht2_cool_off, p_ps_def):
    params = (p_pb_def, p_def_def, p_def_nw, p_def_w,
              p_ht1_tt_def, p_ht1_tt_off,
              p_ht1_cool_def, p_ht1_cool_on, p_ht1_cool_off,
              p_ht2_tt_def, p_ht2_tt_off,
              p_ht2_cool_def, p_ht2_cool_on, p_ht2_cool_off, p_ps_def)
    params2d = [p[None, :] for p in params]

    grid = (B // RB,)
    row_spec9 = pl.BlockSpec((RB, 9), lambda i: (i, 0))
    row_spec6 = pl.BlockSpec((RB, 6), lambda i: (i, 0))
    par_spec = [pl.BlockSpec(p.shape, lambda i: (0, 0)) for p in params2d]

    pb_out, ps_out = pl.pallas_call(
        _body,
        grid=grid,
        in_specs=[row_spec9, row_spec6, row_spec9, row_spec6] + par_spec,
        out_specs=[row_spec9, row_spec6],
        out_shape=[jax.ShapeDtypeStruct((B, 9), jnp.float32),
                   jax.ShapeDtypeStruct((B, 6), jnp.float32)],
    )(proc_bool, proc_scalar, proc_bool_mask, proc_scalar_mask, *params2d)
    return (pb_out, ps_out)


# speedup vs baseline: 3.4996x; 3.4996x over previous
import jax
import jax.numpy as jnp
from jax.experimental import pallas as pl
from jax.experimental.pallas import tpu as pltpu

B = 16384
CB = 2048


def _body(pb_ref, ps_ref, pbm_ref, psm_ref,
          d_pb_ref, d_def_ref, d_nw_ref, d_w_ref,
          d_h1tt_ref, d_h1tt_off_ref, d_h1c_ref, d_h1c_on_ref, d_h1c_off_ref,
          d_h2tt_ref, d_h2tt_off_ref, d_h2c_ref, d_h2c_on_ref, d_h2c_off_ref,
          d_ps_ref, pb_out_ref, ps_out_ref):
    pb = pb_ref[...]
    ps = ps_ref[...]
    pbm = pbm_ref[...]
    psm = psm_ref[...]

    pb1 = pb * pbm + (1.0 - pbm) * d_pb_ref[...]

    cond_nw = (pbm[0:1, :] > 0.5) & (pb1[0:1, :] > 0.5)
    cond_w = (pbm[1:2, :] > 0.5) & (pb1[1:2, :] > 0.5)
    ht1_known = pbm[2:3, :] > 0.5
    ht1_hot = pb1[2:3, :] > 0.5
    ht1_on = ht1_known & ht1_hot
    ht1_off = ht1_known & (~ht1_hot)
    ht2_known = pbm[6:7, :] > 0.5
    ht2_hot = pb1[6:7, :] > 0.5
    ht2_on = ht2_known & ht2_hot
    ht2_off = ht2_known & (~ht2_hot)

    def_fill = jnp.where(cond_w, d_w_ref[...],
                         jnp.where(cond_nw, d_nw_ref[...], d_def_ref[...]))
    ht1_tt = jnp.where(ht1_off, d_h1tt_off_ref[...], d_h1tt_ref[...])
    ht2_tt = jnp.where(ht2_off, d_h2tt_off_ref[...], d_h2tt_ref[...])
    ht1_cool = jnp.where(ht1_off, d_h1c_off_ref[...],
                         jnp.where(ht1_on, d_h1c_on_ref[...], d_h1c_ref[...]))
    ht2_cool = jnp.where(ht2_off, d_h2c_off_ref[...],
                         jnp.where(ht2_on, d_h2c_on_ref[...], d_h2c_ref[...]))

    pb_out_ref[0:3, :] = pb1[0:3, :]
    m36 = pbm[3:6, :]
    pb_out_ref[3:6, :] = pb1[3:6, :] * m36 + (1.0 - m36) * ht1_cool
    pb_out_ref[6:7, :] = pb1[6:7, :]
    m79 = pbm[7:9, :]
    pb_out_ref[7:9, :] = pb1[7:9, :] * m79 + (1.0 - m79) * ht2_cool

    fill_ps = jnp.concatenate([def_fill, ht1_tt, ht2_tt], axis=0)
    t = ps * psm + (1.0 - psm) * fill_ps
    ps_out_ref[...] = t * psm + (1.0 - psm) * d_ps_ref[...]


def kernel(proc_bool, proc_scalar, proc_bool_mask, proc_scalar_mask,
           p_pb_def, p_def_def, p_def_nw, p_def_w,
           p_ht1_tt_def, p_ht1_tt_off,
           p_ht1_cool_def, p_ht1_cool_on, p_ht1_cool_off,
           p_ht2_tt_def, p_ht2_tt_off,
           p_ht2_cool_def, p_ht2_cool_on, p_ht2_cool_off, p_ps_def):
    params = (p_pb_def, p_def_def, p_def_nw, p_def_w,
              p_ht1_tt_def, p_ht1_tt_off,
              p_ht1_cool_def, p_ht1_cool_on, p_ht1_cool_off,
              p_ht2_tt_def, p_ht2_tt_off,
              p_ht2_cool_def, p_ht2_cool_on, p_ht2_cool_off, p_ps_def)
    params2d = [p[:, None] for p in params]

    grid = (B // CB,)
    col_spec9 = pl.BlockSpec((9, CB), lambda i: (0, i))
    col_spec6 = pl.BlockSpec((6, CB), lambda i: (0, i))
    par_spec = [pl.BlockSpec(p.shape, lambda i: (0, 0)) for p in params2d]

    pbT, psT, pbmT, psmT = (proc_bool.T, proc_scalar.T,
                            proc_bool_mask.T, proc_scalar_mask.T)
    pb_out, ps_out = pl.pallas_call(
        _body,
        grid=grid,
        in_specs=[col_spec9, col_spec6, col_spec9, col_spec6] + par_spec,
        out_specs=[col_spec9, col_spec6],
        out_shape=[jax.ShapeDtypeStruct((9, B), jnp.float32),
                   jax.ShapeDtypeStruct((6, B), jnp.float32)],
        compiler_params=pltpu.CompilerParams(
            dimension_semantics=("parallel",)),
    )(pbT, psT, pbmT, psmT, *params2d)
    return (pb_out.T, ps_out.T)


# speedup vs baseline: 7.0146x; 2.0044x over previous
import jax
import jax.numpy as jnp
from jax.experimental import pallas as pl
from jax.experimental.pallas import tpu as pltpu

B = 16384
CB = 2048


def _body(pb_ref, ps_ref, pbm_ref, psm_ref, prm_ref, pb_out_ref, ps_out_ref):
    pb = pb_ref[...]
    ps = ps_ref[...]
    pbm = pbm_ref[...]
    psm = psm_ref[...]
    prm = prm_ref[...]

    d_pb = prm[0:9, :]
    d_def, d_nw, d_w = prm[9:11, :], prm[11:13, :], prm[13:15, :]
    d_h1tt, d_h1tt_off = prm[15:17, :], prm[17:19, :]
    d_h1c, d_h1c_on, d_h1c_off = prm[19:22, :], prm[22:25, :], prm[25:28, :]
    d_h2tt, d_h2tt_off = prm[28:30, :], prm[30:32, :]
    d_h2c, d_h2c_on, d_h2c_off = prm[32:34, :], prm[34:36, :], prm[36:38, :]
    d_ps = prm[38:44, :]

    pb1 = pb * pbm + (1.0 - pbm) * d_pb

    cond_nw = (pbm[0:1, :] > 0.5) & (pb1[0:1, :] > 0.5)
    cond_w = (pbm[1:2, :] > 0.5) & (pb1[1:2, :] > 0.5)
    ht1_known = pbm[2:3, :] > 0.5
    ht1_hot = pb1[2:3, :] > 0.5
    ht1_on = ht1_known & ht1_hot
    ht1_off = ht1_known & (~ht1_hot)
    ht2_known = pbm[6:7, :] > 0.5
    ht2_hot = pb1[6:7, :] > 0.5
    ht2_on = ht2_known & ht2_hot
    ht2_off = ht2_known & (~ht2_hot)

    def_fill = jnp.where(cond_w, d_w, jnp.where(cond_nw, d_nw, d_def))
    ht1_tt = jnp.where(ht1_off, d_h1tt_off, d_h1tt)
    ht2_tt = jnp.where(ht2_off, d_h2tt_off, d_h2tt)
    ht1_cool = jnp.where(ht1_off, d_h1c_off,
                         jnp.where(ht1_on, d_h1c_on, d_h1c))
    ht2_cool = jnp.where(ht2_off, d_h2c_off,
                         jnp.where(ht2_on, d_h2c_on, d_h2c))

    pb_out_ref[0:3, :] = pb1[0:3, :]
    m36 = pbm[3:6, :]
    pb_out_ref[3:6, :] = pb1[3:6, :] * m36 + (1.0 - m36) * ht1_cool
    pb_out_ref[6:7, :] = pb1[6:7, :]
    m79 = pbm[7:9, :]
    pb_out_ref[7:9, :] = pb1[7:9, :] * m79 + (1.0 - m79) * ht2_cool

    fill_ps = jnp.concatenate([def_fill, ht1_tt, ht2_tt], axis=0)
    t = ps * psm + (1.0 - psm) * fill_ps
    ps_out_ref[...] = t * psm + (1.0 - psm) * d_ps


def kernel(proc_bool, proc_scalar, proc_bool_mask, proc_scalar_mask,
           p_pb_def, p_def_def, p_def_nw, p_def_w,
           p_ht1_tt_def, p_ht1_tt_off,
           p_ht1_cool_def, p_ht1_cool_on, p_ht1_cool_off,
           p_ht2_tt_def, p_ht2_tt_off,
           p_ht2_cool_def, p_ht2_cool_on, p_ht2_cool_off, p_ps_def):
    prm = jnp.concatenate(
        [p_pb_def, p_def_def, p_def_nw, p_def_w,
         p_ht1_tt_def, p_ht1_tt_off,
         p_ht1_cool_def, p_ht1_cool_on, p_ht1_cool_off,
         p_ht2_tt_def, p_ht2_tt_off,
         p_ht2_cool_def, p_ht2_cool_on, p_ht2_cool_off, p_ps_def])[:, None]

    grid = (B // CB,)
    col_spec9 = pl.BlockSpec((9, CB), lambda i: (0, i))
    col_spec6 = pl.BlockSpec((6, CB), lambda i: (0, i))
    prm_spec = pl.BlockSpec((44, 1), lambda i: (0, 0))

    pb_out, ps_out = pl.pallas_call(
        _body,
        grid=grid,
        in_specs=[col_spec9, col_spec6, col_spec9, col_spec6, prm_spec],
        out_specs=[col_spec9, col_spec6],
        out_shape=[jax.ShapeDtypeStruct((9, B), jnp.float32),
                   jax.ShapeDtypeStruct((6, B), jnp.float32)],
        compiler_params=pltpu.CompilerParams(
            dimension_semantics=("parallel",)),
    )(proc_bool.T, proc_scalar.T, proc_bool_mask.T, proc_scalar_mask.T, prm)
    return (pb_out.T, ps_out.T)


# speedup vs baseline: 8.7522x; 1.2477x over previous
import jax
import jax.numpy as jnp
from jax.experimental import pallas as pl
from jax.experimental.pallas import tpu as pltpu

B = 16384
CB = 4096


def _body(pb_ref, ps_ref, pbm_ref, psm_ref, prm_ref, pb_out_ref, ps_out_ref):
    pb = pb_ref[...]
    ps = ps_ref[...]
    pbm = pbm_ref[...]
    psm = psm_ref[...]
    prm = prm_ref[...]

    d_pb = prm[0:9, :]
    d_def, d_nw, d_w = prm[9:11, :], prm[11:13, :], prm[13:15, :]
    d_h1tt, d_h1tt_off = prm[15:17, :], prm[17:19, :]
    d_h1c, d_h1c_on, d_h1c_off = prm[19:22, :], prm[22:25, :], prm[25:28, :]
    d_h2tt, d_h2tt_off = prm[28:30, :], prm[30:32, :]
    d_h2c, d_h2c_on, d_h2c_off = prm[32:34, :], prm[34:36, :], prm[36:38, :]
    d_ps = prm[38:44, :]

    pb1 = pb * pbm + (1.0 - pbm) * d_pb

    cond_nw = (pbm[0:1, :] > 0.5) & (pb1[0:1, :] > 0.5)
    cond_w = (pbm[1:2, :] > 0.5) & (pb1[1:2, :] > 0.5)
    ht1_known = pbm[2:3, :] > 0.5
    ht1_hot = pb1[2:3, :] > 0.5
    ht1_on = ht1_known & ht1_hot
    ht1_off = ht1_known & (~ht1_hot)
    ht2_known = pbm[6:7, :] > 0.5
    ht2_hot = pb1[6:7, :] > 0.5
    ht2_on = ht2_known & ht2_hot
    ht2_off = ht2_known & (~ht2_hot)

    def_fill = jnp.where(cond_w, d_w, jnp.where(cond_nw, d_nw, d_def))
    ht1_tt = jnp.where(ht1_off, d_h1tt_off, d_h1tt)
    ht2_tt = jnp.where(ht2_off, d_h2tt_off, d_h2tt)
    ht1_cool = jnp.where(ht1_off, d_h1c_off,
                         jnp.where(ht1_on, d_h1c_on, d_h1c))
    ht2_cool = jnp.where(ht2_off, d_h2c_off,
                         jnp.where(ht2_on, d_h2c_on, d_h2c))

    pb_out_ref[0:3, :] = pb1[0:3, :]
    m36 = pbm[3:6, :]
    pb_out_ref[3:6, :] = pb1[3:6, :] * m36 + (1.0 - m36) * ht1_cool
    pb_out_ref[6:7, :] = pb1[6:7, :]
    m79 = pbm[7:9, :]
    pb_out_ref[7:9, :] = pb1[7:9, :] * m79 + (1.0 - m79) * ht2_cool

    fill_ps = jnp.concatenate([def_fill, ht1_tt, ht2_tt], axis=0)
    t = ps * psm + (1.0 - psm) * fill_ps
    ps_out_ref[...] = t * psm + (1.0 - psm) * d_ps


def kernel(proc_bool, proc_scalar, proc_bool_mask, proc_scalar_mask,
           p_pb_def, p_def_def, p_def_nw, p_def_w,
           p_ht1_tt_def, p_ht1_tt_off,
           p_ht1_cool_def, p_ht1_cool_on, p_ht1_cool_off,
           p_ht2_tt_def, p_ht2_tt_off,
           p_ht2_cool_def, p_ht2_cool_on, p_ht2_cool_off, p_ps_def):
    prm = jnp.concatenate(
        [p_pb_def, p_def_def, p_def_nw, p_def_w,
         p_ht1_tt_def, p_ht1_tt_off,
         p_ht1_cool_def, p_ht1_cool_on, p_ht1_cool_off,
         p_ht2_tt_def, p_ht2_tt_off,
         p_ht2_cool_def, p_ht2_cool_on, p_ht2_cool_off, p_ps_def])[:, None]

    grid = (B // CB,)
    col_spec9 = pl.BlockSpec((9, CB), lambda i: (0, i))
    col_spec6 = pl.BlockSpec((6, CB), lambda i: (0, i))
    prm_spec = pl.BlockSpec((44, 1), lambda i: (0, 0))

    pb_out, ps_out = pl.pallas_call(
        _body,
        grid=grid,
        in_specs=[col_spec9, col_spec6, col_spec9, col_spec6, prm_spec],
        out_specs=[col_spec9, col_spec6],
        out_shape=[jax.ShapeDtypeStruct((9, B), jnp.float32),
                   jax.ShapeDtypeStruct((6, B), jnp.float32)],
        compiler_params=pltpu.CompilerParams(
            dimension_semantics=("parallel",)),
    )(proc_bool.T, proc_scalar.T, proc_bool_mask.T, proc_scalar_mask.T, prm)
    return (pb_out.T, ps_out.T)


# speedup vs baseline: 9.2568x; 1.0577x over previous
import jax
import jax.numpy as jnp
from jax.experimental import pallas as pl
from jax.experimental.pallas import tpu as pltpu

B = 16384
CB = 8192


def _body(pb_ref, ps_ref, pbm_ref, psm_ref, prm_ref, pb_out_ref, ps_out_ref):
    pb = pb_ref[...]
    ps = ps_ref[...]
    pbm = pbm_ref[...]
    psm = psm_ref[...]
    prm = prm_ref[...]

    d_pb = prm[0:9, :]
    d_def, d_nw, d_w = prm[9:11, :], prm[11:13, :], prm[13:15, :]
    d_h1tt, d_h1tt_off = prm[15:17, :], prm[17:19, :]
    d_h1c, d_h1c_on, d_h1c_off = prm[19:22, :], prm[22:25, :], prm[25:28, :]
    d_h2tt, d_h2tt_off = prm[28:30, :], prm[30:32, :]
    d_h2c, d_h2c_on, d_h2c_off = prm[32:34, :], prm[34:36, :], prm[36:38, :]
    d_ps = prm[38:44, :]

    pb1 = pb * pbm + (1.0 - pbm) * d_pb

    cond_nw = (pbm[0:1, :] > 0.5) & (pb1[0:1, :] > 0.5)
    cond_w = (pbm[1:2, :] > 0.5) & (pb1[1:2, :] > 0.5)
    ht1_known = pbm[2:3, :] > 0.5
    ht1_hot = pb1[2:3, :] > 0.5
    ht1_on = ht1_known & ht1_hot
    ht1_off = ht1_known & (~ht1_hot)
    ht2_known = pbm[6:7, :] > 0.5
    ht2_hot = pb1[6:7, :] > 0.5
    ht2_on = ht2_known & ht2_hot
    ht2_off = ht2_known & (~ht2_hot)

    def_fill = jnp.where(cond_w, d_w, jnp.where(cond_nw, d_nw, d_def))
    ht1_tt = jnp.where(ht1_off, d_h1tt_off, d_h1tt)
    ht2_tt = jnp.where(ht2_off, d_h2tt_off, d_h2tt)
    ht1_cool = jnp.where(ht1_off, d_h1c_off,
                         jnp.where(ht1_on, d_h1c_on, d_h1c))
    ht2_cool = jnp.where(ht2_off, d_h2c_off,
                         jnp.where(ht2_on, d_h2c_on, d_h2c))

    pb_out_ref[0:3, :] = pb1[0:3, :]
    m36 = pbm[3:6, :]
    pb_out_ref[3:6, :] = pb1[3:6, :] * m36 + (1.0 - m36) * ht1_cool
    pb_out_ref[6:7, :] = pb1[6:7, :]
    m79 = pbm[7:9, :]
    pb_out_ref[7:9, :] = pb1[7:9, :] * m79 + (1.0 - m79) * ht2_cool

    fill_ps = jnp.concatenate([def_fill, ht1_tt, ht2_tt], axis=0)
    t = ps * psm + (1.0 - psm) * fill_ps
    ps_out_ref[...] = t * psm + (1.0 - psm) * d_ps


def kernel(proc_bool, proc_scalar, proc_bool_mask, proc_scalar_mask,
           p_pb_def, p_def_def, p_def_nw, p_def_w,
           p_ht1_tt_def, p_ht1_tt_off,
           p_ht1_cool_def, p_ht1_cool_on, p_ht1_cool_off,
           p_ht2_tt_def, p_ht2_tt_off,
           p_ht2_cool_def, p_ht2_cool_on, p_ht2_cool_off, p_ps_def):
    prm = jnp.concatenate(
        [p_pb_def, p_def_def, p_def_nw, p_def_w,
         p_ht1_tt_def, p_ht1_tt_off,
         p_ht1_cool_def, p_ht1_cool_on, p_ht1_cool_off,
         p_ht2_tt_def, p_ht2_tt_off,
         p_ht2_cool_def, p_ht2_cool_on, p_ht2_cool_off, p_ps_def])[:, None]

    grid = (B // CB,)
    col_spec9 = pl.BlockSpec((9, CB), lambda i: (0, i))
    col_spec6 = pl.BlockSpec((6, CB), lambda i: (0, i))
    prm_spec = pl.BlockSpec((44, 1), lambda i: (0, 0))

    pb_out, ps_out = pl.pallas_call(
        _body,
        grid=grid,
        in_specs=[col_spec9, col_spec6, col_spec9, col_spec6, prm_spec],
        out_specs=[col_spec9, col_spec6],
        out_shape=[jax.ShapeDtypeStruct((9, B), jnp.float32),
                   jax.ShapeDtypeStruct((6, B), jnp.float32)],
        compiler_params=pltpu.CompilerParams(
            dimension_semantics=("parallel",)),
    )(proc_bool.T, proc_scalar.T, proc_bool_mask.T, proc_scalar_mask.T, prm)
    return (pb_out.T, ps_out.T)
